# baseline (speedup 1.0000x reference)
import functools

import jax
import jax.numpy as jnp
from jax import lax
from jax.experimental import pallas as pl
from jax.experimental.pallas import tpu as pltpu
from jax.experimental.pallas import tpu_sc as plsc

_L = 16

_MESH = plsc.VectorSubcoreMesh(
    core_axis_name="c", subcore_axis_name="s", num_cores=1, num_subcores=1
)


@functools.partial(
    pl.kernel,
    out_type=jax.ShapeDtypeStruct((1,), jnp.float32),
    mesh=_MESH,
    compiler_params=pltpu.CompilerParams(
        needs_layout_passes=False,
        skip_device_barrier=True,
        disable_bounds_checks=True,
        disable_semaphore_checks=True,
    ),
    scratch_types=[
        pltpu.VMEM((48,), jnp.float32),
        pltpu.VMEM((8,), jnp.float32),
        pltpu.VMEM((_L,), jnp.int32),
        pltpu.VMEM((_L,), jnp.float32),
        pltpu.SemaphoreType.DMA,
        pltpu.SemaphoreType.DMA,
        pltpu.SemaphoreType.DMA,
    ],
)
def _sc_row_dot(m_hbm, t_hbm, s_hbm, out_hbm, m_v, t_v, s_v, o_v,
                sem_m, sem_t, sem_s):
    cp_m = pltpu.async_copy(m_hbm, m_v.at[pl.ds(0, 42)], sem_m)
    cp_t = pltpu.async_copy(t_hbm, t_v.at[pl.ds(0, 7)], sem_t)
    cp_s = pltpu.async_copy(s_hbm, s_v.at[pl.ds(0, 1)], sem_s)
    cp_m.wait()
    cp_t.wait()
    cp_s.wait()
    s = s_v[...][0]
    lanes = lax.iota(jnp.int32, _L)
    row = plsc.load_gather(m_v, [jnp.minimum(s * 7 + lanes, 41)])
    tv = plsc.load_gather(t_v, [jnp.minimum(lanes, 6)])
    prod = jnp.where(lanes < 7, row * tv, 0.0)
    o_v[...] = jnp.full((_L,), jnp.sum(prod), jnp.float32)
    pltpu.sync_copy(o_v.at[pl.ds(0, 1)], out_hbm)


def kernel(state, M, theta):
    s_arr = jnp.asarray(state, jnp.int32).reshape(1)
    out = _sc_row_dot(M.reshape(42), theta, s_arr)
    return out.reshape(())


# speedup vs baseline: 1.0869x; 1.0869x over previous
import functools

import jax
import jax.numpy as jnp
from jax import lax
from jax.experimental import pallas as pl
from jax.experimental.pallas import tpu as pltpu
from jax.experimental.pallas import tpu_sc as plsc

_SMESH = plsc.ScalarSubcoreMesh(axis_name="c", num_cores=1)


@functools.partial(
    pl.kernel,
    out_type=jax.ShapeDtypeStruct((1,), jnp.float32),
    mesh=_SMESH,
    compiler_params=pltpu.CompilerParams(needs_layout_passes=False),
    scratch_types=[
        pltpu.SMEM((42,), jnp.float32),
        pltpu.SMEM((7,), jnp.float32),
        pltpu.SMEM((1,), jnp.int32),
        pltpu.SMEM((1,), jnp.float32),
        pltpu.SemaphoreType.DMA,
        pltpu.SemaphoreType.DMA,
        pltpu.SemaphoreType.DMA,
    ],
)
def _scs_row_dot(m_hbm, t_hbm, s_hbm, out_hbm, m_s, t_s, s_s, o_s,
                 sem_m, sem_t, sem_s):
    cp_m = pltpu.async_copy(m_hbm, m_s, sem_m)
    cp_t = pltpu.async_copy(t_hbm, t_s, sem_t)
    cp_s = pltpu.async_copy(s_hbm, s_s, sem_s)
    cp_m.wait()
    cp_t.wait()
    cp_s.wait()
    base = s_s[0] * 7
    acc = m_s[base] * t_s[0]
    for j in range(1, 7):
        acc = acc + m_s[base + j] * t_s[j]
    o_s[0] = acc
    pltpu.sync_copy(o_s, out_hbm)


def kernel(state, M, theta):
    s_arr = jnp.asarray(state, jnp.int32).reshape(1)
    out = _scs_row_dot(M.reshape(42), theta, s_arr)
    return out.reshape(())


# speedup vs baseline: 1.1005x; 1.0126x over previous
import functools

import jax
import jax.numpy as jnp
from jax import lax
from jax.experimental import pallas as pl
from jax.experimental.pallas import tpu as pltpu
from jax.experimental.pallas import tpu_sc as plsc

_SMESH = plsc.ScalarSubcoreMesh(axis_name="c", num_cores=1)


@functools.partial(
    pl.kernel,
    out_type=jax.ShapeDtypeStruct((1,), jnp.float32),
    mesh=_SMESH,
    compiler_params=pltpu.CompilerParams(needs_layout_passes=False),
    scratch_types=[
        pltpu.SMEM((50,), jnp.float32),
        pltpu.SMEM((1,), jnp.float32),
    ],
)
def _scs_row_dot(in_hbm, out_hbm, buf, o_s):
    pltpu.sync_copy(in_hbm, buf)
    base = buf[49].astype(jnp.int32) * 7
    acc = buf[base] * buf[42]
    for j in range(1, 7):
        acc = acc + buf[base + j] * buf[42 + j]
    o_s[0] = acc
    pltpu.sync_copy(o_s, out_hbm)


def kernel(state, M, theta):
    packed = jnp.concatenate(
        [M.reshape(42), theta, jnp.asarray(state, jnp.float32).reshape(1)]
    )
    out = _scs_row_dot(packed)
    return out.reshape(())


# speedup vs baseline: 1.1058x; 1.0048x over previous
import functools

import jax
import jax.numpy as jnp
from jax import lax
from jax.experimental import pallas as pl
from jax.experimental.pallas import tpu as pltpu
from jax.experimental.pallas import tpu_sc as plsc

_SMESH = plsc.ScalarSubcoreMesh(axis_name="c", num_cores=1)


@functools.partial(
    pl.kernel,
    out_type=jax.ShapeDtypeStruct((1,), jnp.float32),
    mesh=_SMESH,
    compiler_params=pltpu.CompilerParams(needs_layout_passes=False),
    scratch_types=[
        pltpu.SMEM((42,), jnp.float32),
        pltpu.SMEM((7,), jnp.float32),
        pltpu.SMEM((1,), jnp.int32),
        pltpu.SMEM((1,), jnp.float32),
        pltpu.SemaphoreType.DMA,
        pltpu.SemaphoreType.DMA,
        pltpu.SemaphoreType.DMA,
    ],
)
def _scs_row_dot(m_hbm, t_hbm, s_hbm, out_hbm, m_s, t_s, s_s, o_s,
                 sem_m, sem_t, sem_s):
    cp_m = pltpu.async_copy(m_hbm, m_s, sem_m)
    cp_t = pltpu.async_copy(t_hbm, t_s, sem_t)
    cp_s = pltpu.async_copy(s_hbm, s_s, sem_s)
    cp_m.wait()
    cp_t.wait()
    cp_s.wait()
    base = s_s[0] * 7
    acc = m_s[base] * t_s[0]
    for j in range(1, 7):
        acc = acc + m_s[base + j] * t_s[j]
    o_s[0] = acc
    pltpu.sync_copy(o_s, out_hbm)


def kernel(state, M, theta):
    s_arr = jnp.asarray(state, jnp.int32).reshape(1)
    out = _scs_row_dot(M.reshape(42), theta, s_arr)
    return out.reshape(())
